# Initial kernel scaffold; baseline (speedup 1.0000x reference)
import jax
import jax.numpy as jnp
from jax.experimental import pallas as pl


def kernel(x, edge_index, W_gat, att_src, att_dst, bias_gat, W_res, b_res, gamma, beta):
    raise NotImplementedError("write your pallas kernel here")


# speedup vs baseline: 2.6289x; 2.6289x over previous
import jax
import jax.numpy as jnp
from jax import lax
from jax.experimental import pallas as pl
from jax.experimental.pallas import tpu as pltpu
from jax.experimental.pallas import tpu_sc as plsc

N = 10000
DIM = 128
H = 4
C = 32
E = 320000
E_TOT = E + N
N_PAD = 10240
HALF = N_PAD // 2
NW = 32
BE = 128
NB = 81
EPW = NB * BE
E_PAD = NW * EPW
SRC_PAD = N_PAD - 1
ROWS_PER_TILE = HALF // 16
DEN_PER_TILE = HALF * H // 16

_f32 = jnp.float32
_i32 = jnp.int32


def _pre_body(x_ref, wg_ref, as_ref, ad_ref, wr_ref, br_ref,
              xl_ref, asrc_ref, adst_ref, res_ref):
    xb = x_ref[...]
    xl = jnp.dot(xb, wg_ref[...], preferred_element_type=_f32)
    xl_ref[...] = xl
    ts = xl * as_ref[...]
    td = xl * ad_ref[...]
    for h in range(H):
        asrc_ref[:, pl.ds(h, 1)] = jnp.sum(ts[:, h * C:(h + 1) * C], axis=1,
                                           keepdims=True)
        adst_ref[:, pl.ds(h, 1)] = jnp.sum(td[:, h * C:(h + 1) * C], axis=1,
                                           keepdims=True)
    res_ref[...] = jnp.dot(xb, wr_ref[...], preferred_element_type=_f32) + br_ref[...]


def _pre_call(xp, W_gat, As, Ad, W_res, b_res):
    blk = N_PAD // 8
    return pl.pallas_call(
        _pre_body,
        grid=(8,),
        in_specs=[
            pl.BlockSpec((blk, DIM), lambda i: (i, 0)),
            pl.BlockSpec((DIM, DIM), lambda i: (0, 0)),
            pl.BlockSpec((DIM,), lambda i: (0,)),
            pl.BlockSpec((DIM,), lambda i: (0,)),
            pl.BlockSpec((DIM, DIM), lambda i: (0, 0)),
            pl.BlockSpec((DIM,), lambda i: (0,)),
        ],
        out_specs=[
            pl.BlockSpec((blk, DIM), lambda i: (i, 0)),
            pl.BlockSpec((blk, H), lambda i: (i, 0)),
            pl.BlockSpec((blk, H), lambda i: (i, 0)),
            pl.BlockSpec((blk, DIM), lambda i: (i, 0)),
        ],
        out_shape=[
            jax.ShapeDtypeStruct((N_PAD, DIM), _f32),
            jax.ShapeDtypeStruct((N_PAD, H), _f32),
            jax.ShapeDtypeStruct((N_PAD, H), _f32),
            jax.ShapeDtypeStruct((N_PAD, DIM), _f32),
        ],
    )(xp, W_gat, As, Ad, W_res, b_res)


def _sc_body(xl_hbm, asrc_hbm, adst_hbm, src_hbm, dst_hbm, zrow_hbm, zden_hbm,
             outp_hbm, denp_hbm,
             src_blk, dst_blk, wflat, idxden, rows_v, asrc_t, adst_t,
             out_acc, den_acc, sem):
    c = lax.axis_index("c")
    s = lax.axis_index("s")
    wid = s * 2 + c
    rb = s * ROWS_PER_TILE
    db = s * DEN_PER_TILE

    pltpu.sync_copy(asrc_hbm, asrc_t)
    iota16 = lax.iota(_i32, 16)

    for p in range(2):
        pltpu.sync_copy(zrow_hbm.at[pl.ds(rb, ROWS_PER_TILE)],
                        out_acc.at[pl.ds(rb, ROWS_PER_TILE)])
        pltpu.sync_copy(zden_hbm.at[pl.ds(db, DEN_PER_TILE)],
                        den_acc.at[pl.ds(db, DEN_PER_TILE)])
        pltpu.sync_copy(adst_hbm.at[pl.ds(p * HALF * H, HALF * H)], adst_t)

        plsc.subcore_barrier()

        def block(b, carry):
            pltpu.sync_copy(src_hbm.at[p, wid, b], src_blk)
            pltpu.sync_copy(dst_hbm.at[p, wid, b], dst_blk)
            for half in range(BE // 16):
                srch = src_blk[0, pl.ds(half * 16, 16)]
                valid = srch < N
                srch4 = srch * 4
                dsth4 = dst_blk[0, pl.ds(half * 16, 16)] * 4
                pos = iota16 + half * 16
                for h in range(H):
                    hs = jnp.full((16,), h, _i32)
                    al = (plsc.load_gather(asrc_t, [srch4 + h])
                          + plsc.load_gather(adst_t, [dsth4 + h]))
                    al = jnp.maximum(al, al * 0.2)
                    wh = jnp.where(valid, jnp.exp(al), 0.0)
                    plsc.store_scatter(wflat, [hs, pos], wh)
                    plsc.store_scatter(idxden, [hs, pos], dsth4 + h)
            for h in range(H):
                pltpu.sync_copy(wflat.at[h], den_acc.at[idxden.at[h]],
                                add=True)
            pltpu.async_copy(xl_hbm.at[src_blk.at[0]], rows_v, sem).wait()
            def scale_one(j, carry):
                js = jnp.full((16,), 0, _i32) + j
                for h in range(H):
                    hs = jnp.full((16,), h, _i32)
                    wjh = plsc.load_gather(wflat, [hs, js])
                    for k2 in range(2):
                        col = h * C + k2 * 16
                        rows_v[j, pl.ds(col, 16)] = (
                            rows_v[j, pl.ds(col, 16)] * wjh)
                return carry

            lax.fori_loop(0, BE, scale_one, 0)
            pltpu.sync_copy(rows_v, out_acc.at[dst_blk.at[0]], add=True)
            return carry

        lax.fori_loop(0, NB, block, 0)

        plsc.subcore_barrier()

        pltpu.sync_copy(out_acc.at[pl.ds(rb, ROWS_PER_TILE)],
                        outp_hbm.at[c, p, pl.ds(rb, ROWS_PER_TILE)])
        pltpu.sync_copy(den_acc.at[pl.ds(db, DEN_PER_TILE)],
                        denp_hbm.at[c, p, pl.ds(db, DEN_PER_TILE)])


def _sc_call(xl, asrc_flat, adst_flat, src5, dst5, zrow, zden_flat):
    mesh = plsc.VectorSubcoreMesh(core_axis_name="c", subcore_axis_name="s",
                                  num_cores=2, num_subcores=16)
    fn = pl.kernel(
        _sc_body,
        out_type=[
            jax.ShapeDtypeStruct((2, 2, HALF, DIM), _f32),
            jax.ShapeDtypeStruct((2, 2, HALF * H), _f32),
        ],
        mesh=mesh,
        compiler_params=pltpu.CompilerParams(needs_layout_passes=False),
        scratch_types=[
            pltpu.VMEM((1, BE), _i32),
            pltpu.VMEM((1, BE), _i32),
            pltpu.VMEM((H, BE), _f32),
            pltpu.VMEM((H, BE), _i32),
            pltpu.VMEM((BE, DIM), _f32),
            pltpu.VMEM((N_PAD * H,), _f32),
            pltpu.VMEM((HALF * H,), _f32),
            pltpu.VMEM_SHARED((HALF, DIM), _f32),
            pltpu.VMEM_SHARED((HALF * H,), _f32),
            pltpu.SemaphoreType.DMA,
        ],
    )
    return fn(xl, asrc_flat, adst_flat, src5, dst5, zrow, zden_flat)


def _post_body(outp_ref, denp_ref, res_ref, bias_ref, g_ref, b_ref,
               o_ref):
    out = outp_ref[0] + outp_ref[1]
    den = denp_ref[0] + denp_ref[1]
    blk = out.shape[0]
    dexp = jnp.concatenate(
        [jnp.broadcast_to(den[:, h:h + 1], (blk, C)) for h in range(H)],
        axis=1)
    hh = out / dexp + bias_ref[...] + res_ref[...]
    mu = jnp.mean(hh, axis=1, keepdims=True)
    dd = hh - mu
    var = jnp.mean(dd * dd, axis=1, keepdims=True)
    o_ref[...] = g_ref[...] * dd * lax.rsqrt(var + 1e-5) + b_ref[...]


def _post_call(outp, denp, res, bias_gat, gamma, beta):
    blk = 1000
    return pl.pallas_call(
        _post_body,
        grid=(N // blk,),
        in_specs=[
            pl.BlockSpec((2, blk, DIM), lambda i: (0, i, 0)),
            pl.BlockSpec((2, blk, H), lambda i: (0, i, 0)),
            pl.BlockSpec((blk, DIM), lambda i: (i, 0)),
            pl.BlockSpec((DIM,), lambda i: (0,)),
            pl.BlockSpec((DIM,), lambda i: (0,)),
            pl.BlockSpec((DIM,), lambda i: (0,)),
        ],
        out_specs=pl.BlockSpec((blk, DIM), lambda i: (i, 0)),
        out_shape=jax.ShapeDtypeStruct((N, DIM), _f32),
    )(outp, denp, res, bias_gat, gamma, beta)


def kernel(x, edge_index, W_gat, att_src, att_dst, bias_gat, W_res, b_res,
           gamma, beta):
    loop = jnp.arange(N, dtype=_i32)
    src = jnp.concatenate([edge_index[0].astype(_i32), loop])
    dst = jnp.concatenate([edge_index[1].astype(_i32), loop])
    hsel = (dst >= HALF).astype(_i32)
    cs1 = jnp.cumsum(hsel)
    idx_all = jnp.arange(E_TOT)
    cs0 = idx_all + 1 - cs1
    pos = jnp.where(hsel == 0, cs0 - 1, E_PAD + cs1 - 1)
    srcb = jnp.full((2 * E_PAD,), SRC_PAD, _i32).at[pos].set(src)
    dstb = jnp.zeros((2 * E_PAD,), _i32).at[pos].set(dst - hsel * HALF)
    src5 = srcb.reshape(2, NW, NB, 1, BE)
    dst5 = dstb.reshape(2, NW, NB, 1, BE)

    xp = jnp.pad(x, ((0, N_PAD - N), (0, 0)))
    zrow = jnp.zeros((HALF, DIM), _f32)
    zden = jnp.zeros((HALF * H,), _f32)

    xl, asrc, adst, res = _pre_call(xp, W_gat, att_src.reshape(-1),
                                    att_dst.reshape(-1), W_res, b_res)
    outp, denp = _sc_call(xl, asrc.reshape(-1), adst.reshape(-1),
                          src5, dst5, zrow, zden)
    return _post_call(outp.reshape(2, N_PAD, DIM),
                      denp.reshape(2, N_PAD, H), res,
                      bias_gat, gamma, beta)
